# baseline (speedup 1.0000x reference)
import jax
import jax.numpy as jnp
from jax.experimental import pallas as pl
from jax.experimental.pallas import tpu as pltpu

_EPS = 1e-5
_INF = 1e5
_BLK = 128


def _silu(t):
    return t * jax.nn.sigmoid(t)


def _softmax_lane(v):
    m = jnp.max(v, axis=-1, keepdims=True)
    e = jnp.exp(v - m)
    return e / jnp.sum(e, axis=-1, keepdims=True)


def _sake_body(h_ref, x_ref, hfull_ref, xfull_ref, w1a_ref, w1b_ref, w1d_ref,
               b1_ref, w2_ref, b2c_ref, semw_ref, sembc_ref, pnw1_ref,
               pnb1_ref, pnw2_ref, pnb2_ref, wnh_ref, wg_ref, wnc_ref,
               nb1_ref, nw2_ref, nb2_ref, vmix_ref, lg_ref,
               hout_ref, xout_ref, hjT_ref, xT_ref):
    n = hfull_ref.shape[0]
    blk = h_ref.shape[0]
    step = pl.program_id(0)

    @pl.when(step == 0)
    def _():
        hj = jnp.dot(hfull_ref[...], w1b_ref[...],
                     preferred_element_type=jnp.float32)
        hjT_ref[...] = hj.T
        xT_ref[...] = xfull_ref[...].T

    h_blk = h_ref[...]
    x_blk = x_ref[...]
    xT = xT_ref[...]

    dx0 = x_blk[:, 0:1] - xT[0:1, :]
    dx1 = x_blk[:, 1:2] - xT[1:2, :]
    dx2 = x_blk[:, 2:3] - xT[2:3, :]
    d2 = dx0 * dx0 + dx1 * dx1 + dx2 * dx2
    dist = jnp.sqrt(d2 + _EPS)

    hi = jnp.dot(h_blk, w1a_ref[...],
                 preferred_element_type=jnp.float32) + b1_ref[...]
    pre = (hi[:, :, None] + hjT_ref[...][None, :, :]
           + dist[:, None, :] * w1d_ref[...][None, :, :])
    t = _silu(pre)

    he_rows = []
    for b in range(blk):
        he_rows.append(jax.lax.dot_general(
            w2_ref[...], t[b], (((0,), (0,)), ((), ())),
            preferred_element_type=jnp.float32))
    h_e = _silu(jnp.stack(he_rows, axis=0) + b2c_ref[...][None, :, :])

    sl_rows = []
    for b in range(blk):
        sl_rows.append(jax.lax.dot_general(
            semw_ref[...], h_e[b], (((0,), (0,)), ((), ())),
            preferred_element_type=jnp.float32))
    s_log = jnp.stack(sl_rows, axis=0) + sembc_ref[...][None, :, :]
    s_log = jnp.where(s_log >= 0, s_log, 0.2 * s_log)

    col = jax.lax.broadcasted_iota(jnp.int32, (blk, n), 1)
    row = jax.lax.broadcasted_iota(jnp.int32, (blk, n), 0) + step * blk
    eye = (col == row).astype(jnp.float32)[:, None, :]

    gamma = jnp.exp(lg_ref[...])[:, :, None]
    dist_m = dist + (_INF * eye)[:, 0, :]
    e_log = -dist_m[:, None, :] * gamma
    s_log = s_log - _INF * eye

    e_exp = jnp.exp(e_log)
    ze = jnp.sum(e_exp, axis=-1, keepdims=True)
    ms = jnp.max(s_log, axis=-1, keepdims=True)
    s_exp = jnp.exp(s_log - ms)
    zs = jnp.sum(s_exp, axis=-1, keepdims=True)
    u = e_exp * s_exp
    comb = u / (jnp.sum(u, axis=-1, keepdims=True) + _EPS * ze * zs)

    w = 1.0 / (dist + _EPS)
    wdx0 = w * dx0
    wdx1 = w * dx1
    wdx2 = w * dx2
    qs_rows = []
    for b in range(blk):
        g_all = jax.lax.dot_general(
            wg_ref[...], h_e[b], (((1,), (0,)), ((), ())),
            preferred_element_type=jnp.float32)
        acc = comb[b, 0:1, :] * g_all[0:32, :]
        for m in range(1, 4):
            acc = acc + comb[b, m:m + 1, :] * g_all[32 * m:32 * (m + 1), :]
        qs_rows.append(acc)
    n1_agg = jnp.sum(jnp.stack(qs_rows, axis=0), axis=-1)
    csn0 = jnp.sum(h_e * wdx0[:, None, :], axis=-1)
    csn1 = jnp.sum(h_e * wdx1[:, None, :], axis=-1)
    csn2 = jnp.sum(h_e * wdx2[:, None, :], axis=-1)

    inv_n = 1.0 / n
    cnorm = (csn0 * csn0 + csn1 * csn1 + csn2 * csn2) * (inv_n * inv_n)
    hc1 = _silu(jnp.dot(cnorm, pnw1_ref[...],
                        preferred_element_type=jnp.float32) + pnb1_ref[...])
    h_comb = _silu(jnp.dot(hc1, pnw2_ref[...],
                           preferred_element_type=jnp.float32) + pnb2_ref[...])

    vm = vmix_ref[...]
    dv0 = jnp.dot(csn0, vm, preferred_element_type=jnp.float32) * inv_n
    dv1 = jnp.dot(csn1, vm, preferred_element_type=jnp.float32) * inv_n
    dv2 = jnp.dot(csn2, vm, preferred_element_type=jnp.float32) * inv_n
    xout_ref[...] = x_blk + jnp.concatenate([dv0, dv1, dv2], axis=1)

    n1 = (jnp.dot(h_blk, wnh_ref[...], preferred_element_type=jnp.float32)
          + jnp.dot(h_comb, wnc_ref[...], preferred_element_type=jnp.float32)
          + n1_agg + nb1_ref[...])
    n1 = _silu(n1)
    hout_ref[...] = h_blk + _silu(
        jnp.dot(n1, nw2_ref[...], preferred_element_type=jnp.float32)
        + nb2_ref[...])


def kernel(h, x, edge_w1, edge_b1, edge_w2, edge_b2, sem_w, sem_b, pn_w1,
           pn_b1, pn_w2, pn_b2, node_w1, node_b1, node_w2, node_b2, vmix_w,
           log_gamma):
    n, d_in = h.shape
    hidden = edge_w2.shape[0]
    heads = sem_w.shape[1]
    d_out = node_w2.shape[1]
    blk = _BLK

    w1a = edge_w1[:d_in]
    w1b = edge_w1[d_in:2 * d_in]
    w1d = edge_w1[2 * d_in:].T
    b1 = edge_b1.reshape(1, hidden)
    b2c = edge_b2.reshape(hidden, 1)
    sembc = sem_b.reshape(heads, 1)
    pnb1 = pn_b1.reshape(1, hidden)
    pnb2 = pn_b2.reshape(1, hidden)
    wnh = node_w1[:d_in]
    wg = (node_w1[d_in:d_in + heads * hidden]
          .reshape(hidden, heads, hidden).transpose(1, 2, 0)
          .reshape(heads * hidden, hidden))
    wnc = node_w1[d_in + heads * hidden:]
    nb1 = node_b1.reshape(1, hidden)
    nb2 = node_b2.reshape(1, d_out)
    lg = log_gamma.reshape(1, heads)

    full = lambda shape: pl.BlockSpec(shape, lambda i: (0,) * len(shape))
    out = pl.pallas_call(
        _sake_body,
        grid=(n // blk,),
        in_specs=[
            pl.BlockSpec((blk, d_in), lambda i: (i, 0)),
            pl.BlockSpec((blk, 3), lambda i: (i, 0)),
            full((n, d_in)),
            full((n, 3)),
            full((d_in, hidden)),
            full((d_in, hidden)),
            full((hidden, 1)),
            full((1, hidden)),
            full((hidden, hidden)),
            full((hidden, 1)),
            full((hidden, heads)),
            full((heads, 1)),
            full((hidden, hidden)),
            full((1, hidden)),
            full((hidden, hidden)),
            full((1, hidden)),
            full((d_in, hidden)),
            full((heads * hidden, hidden)),
            full((hidden, hidden)),
            full((1, hidden)),
            full((hidden, d_out)),
            full((1, d_out)),
            full((hidden, 1)),
            full((1, heads)),
        ],
        out_specs=[
            pl.BlockSpec((blk, d_out), lambda i: (i, 0)),
            pl.BlockSpec((blk, 3), lambda i: (i, 0)),
        ],
        out_shape=[
            jax.ShapeDtypeStruct((n, d_out), jnp.float32),
            jax.ShapeDtypeStruct((n, 3), jnp.float32),
        ],
        scratch_shapes=[pltpu.VMEM((hidden, n), jnp.float32),
                        pltpu.VMEM((3, n), jnp.float32)],
    )(h, x, h, x, w1a, w1b, w1d, b1, edge_w2, b2c, sem_w, sembc,
      pn_w1, pnb1, pn_w2, pnb2, wnh, wg, wnc, nb1, node_w2, nb2,
      vmix_w, lg)
    return out[0], out[1]


# speedup vs baseline: 1.0090x; 1.0090x over previous
import jax
import jax.numpy as jnp
from jax.experimental import pallas as pl
from jax.experimental.pallas import tpu as pltpu

_EPS = 1e-5
_INF = 1e5
_BLK = 128


def _silu(t):
    return t * jax.nn.sigmoid(t)


def _softmax_lane(v):
    m = jnp.max(v, axis=-1, keepdims=True)
    e = jnp.exp(v - m)
    return e / jnp.sum(e, axis=-1, keepdims=True)


def _sake_body(h_ref, x_ref, hfull_ref, xfull_ref, w1a_ref, w1b_ref, w1d_ref,
               b1_ref, w2_ref, b2c_ref, semw_ref, sembc_ref, pnw1_ref,
               pnb1_ref, pnw2_ref, pnb2_ref, wnh_ref, wg_ref, wnc_ref,
               nb1_ref, nw2_ref, nb2_ref, vmix_ref, lg_ref,
               hout_ref, xout_ref, hjT_ref, xT_ref):
    n = hfull_ref.shape[0]
    blk = h_ref.shape[0]
    step = pl.program_id(0)

    hj = jnp.dot(hfull_ref[...], w1b_ref[...],
                 preferred_element_type=jnp.float32)
    hjT_ref[...] = hj.T
    xT_ref[...] = xfull_ref[...].T

    h_blk = h_ref[...]
    x_blk = x_ref[...]
    xT = xT_ref[...]

    dx0 = x_blk[:, 0:1] - xT[0:1, :]
    dx1 = x_blk[:, 1:2] - xT[1:2, :]
    dx2 = x_blk[:, 2:3] - xT[2:3, :]
    d2 = dx0 * dx0 + dx1 * dx1 + dx2 * dx2
    dist = jnp.sqrt(d2 + _EPS)

    hi = jnp.dot(h_blk, w1a_ref[...],
                 preferred_element_type=jnp.float32) + b1_ref[...]
    pre = (hi[:, :, None] + hjT_ref[...][None, :, :]
           + dist[:, None, :] * w1d_ref[...][None, :, :])
    t = _silu(pre)

    he_rows = []
    for b in range(blk):
        he_rows.append(jax.lax.dot_general(
            w2_ref[...], t[b], (((0,), (0,)), ((), ())),
            preferred_element_type=jnp.float32))
    h_e = _silu(jnp.stack(he_rows, axis=0) + b2c_ref[...][None, :, :])

    sl_rows = []
    for b in range(blk):
        sl_rows.append(jax.lax.dot_general(
            semw_ref[...], h_e[b], (((0,), (0,)), ((), ())),
            preferred_element_type=jnp.float32))
    s_log = jnp.stack(sl_rows, axis=0) + sembc_ref[...][None, :, :]
    s_log = jnp.where(s_log >= 0, s_log, 0.2 * s_log)

    col = jax.lax.broadcasted_iota(jnp.int32, (blk, n), 1)
    row = jax.lax.broadcasted_iota(jnp.int32, (blk, n), 0) + step * blk
    eye = (col == row).astype(jnp.float32)[:, None, :]

    gamma = jnp.exp(lg_ref[...])[:, :, None]
    dist_m = dist + (_INF * eye)[:, 0, :]
    e_log = -dist_m[:, None, :] * gamma
    s_log = s_log - _INF * eye

    e_exp = jnp.exp(e_log)
    ze = jnp.sum(e_exp, axis=-1, keepdims=True)
    ms = jnp.max(s_log, axis=-1, keepdims=True)
    s_exp = jnp.exp(s_log - ms)
    zs = jnp.sum(s_exp, axis=-1, keepdims=True)
    u = e_exp * s_exp
    comb = u / (jnp.sum(u, axis=-1, keepdims=True) + _EPS * ze * zs)

    w = 1.0 / (dist + _EPS)
    wdx0 = w * dx0
    wdx1 = w * dx1
    wdx2 = w * dx2
    qs_rows = []
    for b in range(blk):
        g_all = jax.lax.dot_general(
            wg_ref[...], h_e[b], (((1,), (0,)), ((), ())),
            preferred_element_type=jnp.float32)
        acc = comb[b, 0:1, :] * g_all[0:32, :]
        for m in range(1, 4):
            acc = acc + comb[b, m:m + 1, :] * g_all[32 * m:32 * (m + 1), :]
        qs_rows.append(acc)
    n1_agg = jnp.sum(jnp.stack(qs_rows, axis=0), axis=-1)
    csn0 = jnp.sum(h_e * wdx0[:, None, :], axis=-1)
    csn1 = jnp.sum(h_e * wdx1[:, None, :], axis=-1)
    csn2 = jnp.sum(h_e * wdx2[:, None, :], axis=-1)

    inv_n = 1.0 / n
    cnorm = (csn0 * csn0 + csn1 * csn1 + csn2 * csn2) * (inv_n * inv_n)
    hc1 = _silu(jnp.dot(cnorm, pnw1_ref[...],
                        preferred_element_type=jnp.float32) + pnb1_ref[...])
    h_comb = _silu(jnp.dot(hc1, pnw2_ref[...],
                           preferred_element_type=jnp.float32) + pnb2_ref[...])

    vm = vmix_ref[...]
    dv0 = jnp.dot(csn0, vm, preferred_element_type=jnp.float32) * inv_n
    dv1 = jnp.dot(csn1, vm, preferred_element_type=jnp.float32) * inv_n
    dv2 = jnp.dot(csn2, vm, preferred_element_type=jnp.float32) * inv_n
    xout_ref[...] = x_blk + jnp.concatenate([dv0, dv1, dv2], axis=1)

    n1 = (jnp.dot(h_blk, wnh_ref[...], preferred_element_type=jnp.float32)
          + jnp.dot(h_comb, wnc_ref[...], preferred_element_type=jnp.float32)
          + n1_agg + nb1_ref[...])
    n1 = _silu(n1)
    hout_ref[...] = h_blk + _silu(
        jnp.dot(n1, nw2_ref[...], preferred_element_type=jnp.float32)
        + nb2_ref[...])


def kernel(h, x, edge_w1, edge_b1, edge_w2, edge_b2, sem_w, sem_b, pn_w1,
           pn_b1, pn_w2, pn_b2, node_w1, node_b1, node_w2, node_b2, vmix_w,
           log_gamma):
    n, d_in = h.shape
    hidden = edge_w2.shape[0]
    heads = sem_w.shape[1]
    d_out = node_w2.shape[1]
    blk = _BLK

    w1a = edge_w1[:d_in]
    w1b = edge_w1[d_in:2 * d_in]
    w1d = edge_w1[2 * d_in:].T
    b1 = edge_b1.reshape(1, hidden)
    b2c = edge_b2.reshape(hidden, 1)
    sembc = sem_b.reshape(heads, 1)
    pnb1 = pn_b1.reshape(1, hidden)
    pnb2 = pn_b2.reshape(1, hidden)
    wnh = node_w1[:d_in]
    wg = (node_w1[d_in:d_in + heads * hidden]
          .reshape(hidden, heads, hidden).transpose(1, 2, 0)
          .reshape(heads * hidden, hidden))
    wnc = node_w1[d_in + heads * hidden:]
    nb1 = node_b1.reshape(1, hidden)
    nb2 = node_b2.reshape(1, d_out)
    lg = log_gamma.reshape(1, heads)

    full = lambda shape: pl.BlockSpec(shape, lambda i: (0,) * len(shape))
    out = pl.pallas_call(
        _sake_body,
        grid=(n // blk,),
        compiler_params=pltpu.CompilerParams(
            dimension_semantics=("parallel",)),
        in_specs=[
            pl.BlockSpec((blk, d_in), lambda i: (i, 0)),
            pl.BlockSpec((blk, 3), lambda i: (i, 0)),
            full((n, d_in)),
            full((n, 3)),
            full((d_in, hidden)),
            full((d_in, hidden)),
            full((hidden, 1)),
            full((1, hidden)),
            full((hidden, hidden)),
            full((hidden, 1)),
            full((hidden, heads)),
            full((heads, 1)),
            full((hidden, hidden)),
            full((1, hidden)),
            full((hidden, hidden)),
            full((1, hidden)),
            full((d_in, hidden)),
            full((heads * hidden, hidden)),
            full((hidden, hidden)),
            full((1, hidden)),
            full((hidden, d_out)),
            full((1, d_out)),
            full((hidden, 1)),
            full((1, heads)),
        ],
        out_specs=[
            pl.BlockSpec((blk, d_out), lambda i: (i, 0)),
            pl.BlockSpec((blk, 3), lambda i: (i, 0)),
        ],
        out_shape=[
            jax.ShapeDtypeStruct((n, d_out), jnp.float32),
            jax.ShapeDtypeStruct((n, 3), jnp.float32),
        ],
        scratch_shapes=[pltpu.VMEM((hidden, n), jnp.float32),
                        pltpu.VMEM((3, n), jnp.float32)],
    )(h, x, h, x, w1a, w1b, w1d, b1, edge_w2, b2c, sem_w, sembc,
      pn_w1, pnb1, pn_w2, pnb2, wnh, wg, wnc, nb1, node_w2, nb2,
      vmix_w, lg)
    return out[0], out[1]


# speedup vs baseline: 1.0638x; 1.0543x over previous
import jax
import jax.numpy as jnp
from jax.experimental import pallas as pl
from jax.experimental.pallas import tpu as pltpu

_EPS = 1e-5
_INF = 1e5
_BLK = 128


def _silu(t):
    return t * jax.nn.sigmoid(t)


def _softmax_lane(v):
    m = jnp.max(v, axis=-1, keepdims=True)
    e = jnp.exp(v - m)
    return e / jnp.sum(e, axis=-1, keepdims=True)


def _sake_body(h_ref, x_ref, hfull_ref, xfull_ref, ew1_ref,
               b1_ref, w2_ref, b2c_ref, semw_ref, sembc_ref, pnw1_ref,
               pnb1_ref, pnw2_ref, pnb2_ref, nw1_ref, wg_ref,
               nb1_ref, nw2_ref, nb2_ref, vmix_ref, lg_ref,
               hout_ref, xout_ref, hjT_ref, xT_ref):
    n = hfull_ref.shape[0]
    d_in = hfull_ref.shape[1]
    blk = h_ref.shape[0]
    step = pl.program_id(0)

    hj = jnp.dot(hfull_ref[...], ew1_ref[d_in:2 * d_in, :],
                 preferred_element_type=jnp.float32)
    hjT_ref[...] = hj.T
    xT_ref[...] = xfull_ref[...].T

    h_blk = h_ref[...]
    x_blk = x_ref[...]
    xT = xT_ref[...]

    dx0 = x_blk[:, 0:1] - xT[0:1, :]
    dx1 = x_blk[:, 1:2] - xT[1:2, :]
    dx2 = x_blk[:, 2:3] - xT[2:3, :]
    d2 = dx0 * dx0 + dx1 * dx1 + dx2 * dx2
    dist = jnp.sqrt(d2 + _EPS)

    hi = jnp.dot(h_blk, ew1_ref[0:d_in, :],
                 preferred_element_type=jnp.float32) + b1_ref[...]
    w1d = ew1_ref[2 * d_in:2 * d_in + 1, :].T
    pre = (hi[:, :, None] + hjT_ref[...][None, :, :]
           + dist[:, None, :] * w1d[None, :, :])
    t = _silu(pre)

    he_rows = []
    for b in range(blk):
        he_rows.append(jax.lax.dot_general(
            w2_ref[...], t[b], (((0,), (0,)), ((), ())),
            preferred_element_type=jnp.float32))
    h_e = _silu(jnp.stack(he_rows, axis=0) + b2c_ref[...][None, :, :])

    sl_rows = []
    for b in range(blk):
        sl_rows.append(jax.lax.dot_general(
            semw_ref[...], h_e[b], (((0,), (0,)), ((), ())),
            preferred_element_type=jnp.float32))
    s_log = jnp.stack(sl_rows, axis=0) + sembc_ref[...][None, :, :]
    s_log = jnp.where(s_log >= 0, s_log, 0.2 * s_log)

    col = jax.lax.broadcasted_iota(jnp.int32, (blk, n), 1)
    row = jax.lax.broadcasted_iota(jnp.int32, (blk, n), 0) + step * blk
    eye = (col == row).astype(jnp.float32)[:, None, :]

    gamma = jnp.exp(lg_ref[...])[:, :, None]
    dist_m = dist + (_INF * eye)[:, 0, :]
    e_log = -dist_m[:, None, :] * gamma
    s_log = s_log - _INF * eye

    e_exp = jnp.exp(e_log)
    ze = jnp.sum(e_exp, axis=-1, keepdims=True)
    ms = jnp.max(s_log, axis=-1, keepdims=True)
    s_exp = jnp.exp(s_log - ms)
    zs = jnp.sum(s_exp, axis=-1, keepdims=True)
    u = e_exp * s_exp
    comb = u / (jnp.sum(u, axis=-1, keepdims=True) + _EPS * ze * zs)

    w = 1.0 / (dist + _EPS)
    wdx0 = w * dx0
    wdx1 = w * dx1
    wdx2 = w * dx2
    qs_rows = []
    for b in range(blk):
        g_all = jax.lax.dot_general(
            wg_ref[...], h_e[b], (((1,), (0,)), ((), ())),
            preferred_element_type=jnp.float32)
        acc = comb[b, 0:1, :] * g_all[0:32, :]
        for m in range(1, 4):
            acc = acc + comb[b, m:m + 1, :] * g_all[32 * m:32 * (m + 1), :]
        qs_rows.append(acc)
    n1_agg = jnp.sum(jnp.stack(qs_rows, axis=0), axis=-1)
    csn0 = jnp.sum(h_e * wdx0[:, None, :], axis=-1)
    csn1 = jnp.sum(h_e * wdx1[:, None, :], axis=-1)
    csn2 = jnp.sum(h_e * wdx2[:, None, :], axis=-1)

    inv_n = 1.0 / n
    cnorm = (csn0 * csn0 + csn1 * csn1 + csn2 * csn2) * (inv_n * inv_n)
    hc1 = _silu(jnp.dot(cnorm, pnw1_ref[...],
                        preferred_element_type=jnp.float32) + pnb1_ref[...])
    h_comb = _silu(jnp.dot(hc1, pnw2_ref[...],
                           preferred_element_type=jnp.float32) + pnb2_ref[...])

    vm = vmix_ref[...]
    dv0 = jnp.dot(csn0, vm, preferred_element_type=jnp.float32) * inv_n
    dv1 = jnp.dot(csn1, vm, preferred_element_type=jnp.float32) * inv_n
    dv2 = jnp.dot(csn2, vm, preferred_element_type=jnp.float32) * inv_n
    xout_ref[...] = x_blk + jnp.concatenate([dv0, dv1, dv2], axis=1)

    hvh = nw1_ref.shape[0]
    n1 = (jnp.dot(h_blk, nw1_ref[0:d_in, :],
                  preferred_element_type=jnp.float32)
          + jnp.dot(h_comb, nw1_ref[hvh - 32:hvh, :],
                    preferred_element_type=jnp.float32)
          + n1_agg + nb1_ref[...])
    n1 = _silu(n1)
    hout_ref[...] = h_blk + _silu(
        jnp.dot(n1, nw2_ref[...], preferred_element_type=jnp.float32)
        + nb2_ref[...])


def kernel(h, x, edge_w1, edge_b1, edge_w2, edge_b2, sem_w, sem_b, pn_w1,
           pn_b1, pn_w2, pn_b2, node_w1, node_b1, node_w2, node_b2, vmix_w,
           log_gamma):
    n, d_in = h.shape
    hidden = edge_w2.shape[0]
    heads = sem_w.shape[1]
    d_out = node_w2.shape[1]
    blk = _BLK

    b1 = edge_b1.reshape(1, hidden)
    b2c = edge_b2.reshape(hidden, 1)
    sembc = sem_b.reshape(heads, 1)
    pnb1 = pn_b1.reshape(1, hidden)
    pnb2 = pn_b2.reshape(1, hidden)
    wg = (node_w1[d_in:d_in + heads * hidden]
          .reshape(hidden, heads, hidden).transpose(1, 2, 0)
          .reshape(heads * hidden, hidden))
    nb1 = node_b1.reshape(1, hidden)
    nb2 = node_b2.reshape(1, d_out)
    lg = log_gamma.reshape(1, heads)

    full = lambda shape: pl.BlockSpec(shape, lambda i: (0,) * len(shape))
    out = pl.pallas_call(
        _sake_body,
        grid=(n // blk,),
        compiler_params=pltpu.CompilerParams(
            dimension_semantics=("parallel",)),
        in_specs=[
            pl.BlockSpec((blk, d_in), lambda i: (i, 0)),
            pl.BlockSpec((blk, 3), lambda i: (i, 0)),
            full((n, d_in)),
            full((n, 3)),
            full((2 * d_in + 1, hidden)),
            full((1, hidden)),
            full((hidden, hidden)),
            full((hidden, 1)),
            full((hidden, heads)),
            full((heads, 1)),
            full((hidden, hidden)),
            full((1, hidden)),
            full((hidden, hidden)),
            full((1, hidden)),
            full((heads * hidden + hidden + d_in, hidden)),
            full((heads * hidden, hidden)),
            full((1, hidden)),
            full((hidden, d_out)),
            full((1, d_out)),
            full((hidden, 1)),
            full((1, heads)),
        ],
        out_specs=[
            pl.BlockSpec((blk, d_out), lambda i: (i, 0)),
            pl.BlockSpec((blk, 3), lambda i: (i, 0)),
        ],
        out_shape=[
            jax.ShapeDtypeStruct((n, d_out), jnp.float32),
            jax.ShapeDtypeStruct((n, 3), jnp.float32),
        ],
        scratch_shapes=[pltpu.VMEM((hidden, n), jnp.float32),
                        pltpu.VMEM((3, n), jnp.float32)],
    )(h, x, h, x, edge_w1, b1, edge_w2, b2c, sem_w, sembc,
      pn_w1, pnb1, pn_w2, pnb2, node_w1, wg, nb1, node_w2, nb2,
      vmix_w, lg)
    return out[0], out[1]


# speedup vs baseline: 1.1169x; 1.0498x over previous
import jax
import jax.numpy as jnp
from jax.experimental import pallas as pl
from jax.experimental.pallas import tpu as pltpu

_EPS = 1e-5
_INF = 1e5
_BLK = 128


def _silu(t):
    return t * jax.nn.sigmoid(t)


def _softmax_lane(v):
    m = jnp.max(v, axis=-1, keepdims=True)
    e = jnp.exp(v - m)
    return e / jnp.sum(e, axis=-1, keepdims=True)


def _sake_body(h_ref, x_ref, hfull_ref, xfull_ref, ew1_ref,
               b1_ref, w2_ref, b2c_ref, semw_ref, sembc_ref, pnw1_ref,
               pnb1_ref, pnw2_ref, pnb2_ref, nw1_ref, wg_ref,
               nb1_ref, nw2_ref, nb2_ref, vmix_ref, lg_ref,
               hout_ref, xout_ref, hjT_ref, xT_ref):
    n = hfull_ref.shape[0]
    d_in = hfull_ref.shape[1]
    blk = h_ref.shape[0]
    step = pl.program_id(0)

    hj = jnp.dot(hfull_ref[...], ew1_ref[d_in:2 * d_in, :],
                 preferred_element_type=jnp.float32)
    hjT_ref[...] = hj.T
    xT_ref[...] = xfull_ref[...].T

    h_blk = h_ref[...]
    x_blk = x_ref[...]
    xT = xT_ref[...]

    dx0 = x_blk[:, 0:1] - xT[0:1, :]
    dx1 = x_blk[:, 1:2] - xT[1:2, :]
    dx2 = x_blk[:, 2:3] - xT[2:3, :]
    d2 = dx0 * dx0 + dx1 * dx1 + dx2 * dx2
    dist = jnp.sqrt(d2 + _EPS)

    hi = jnp.dot(h_blk, ew1_ref[0:d_in, :],
                 preferred_element_type=jnp.float32) + b1_ref[...]
    w1d = ew1_ref[2 * d_in:2 * d_in + 1, :].T
    pre = (hi[:, :, None] + hjT_ref[...][None, :, :]
           + dist[:, None, :] * w1d[None, :, :])
    t = _silu(pre)

    he_rows = []
    for b in range(blk):
        he_rows.append(jax.lax.dot_general(
            w2_ref[...], t[b], (((0,), (0,)), ((), ())),
            preferred_element_type=jnp.float32))
    h_e = _silu(jnp.stack(he_rows, axis=0) + b2c_ref[...][None, :, :])

    hp = blk // 2
    sl_rows = []
    for b in range(blk):
        sl_rows.append(jax.lax.dot_general(
            semw_ref[...], h_e[b], (((0,), (0,)), ((), ())),
            preferred_element_type=jnp.float32))
    s_log = jnp.stack(
        [jnp.concatenate([sl_rows[2 * q], sl_rows[2 * q + 1]], axis=0)
         for q in range(hp)], axis=0)
    semb8 = jnp.concatenate([sembc_ref[...], sembc_ref[...]], axis=0)
    s_log = s_log + semb8[None, :, :]
    s_log = jnp.where(s_log >= 0, s_log, 0.2 * s_log)

    col = jax.lax.broadcasted_iota(jnp.int32, (blk, n), 1)
    row = jax.lax.broadcasted_iota(jnp.int32, (blk, n), 0) + step * blk
    eye = (col == row).astype(jnp.float32)
    dist_m = dist + _INF * eye

    colp = jax.lax.broadcasted_iota(jnp.int32, (hp, 8, n), 2)
    rowp = (step * blk
            + 2 * jax.lax.broadcasted_iota(jnp.int32, (hp, 8, n), 0)
            + jax.lax.broadcasted_iota(jnp.int32, (hp, 8, n), 1) // 4)
    eyep = (colp == rowp).astype(jnp.float32)

    lg8 = jnp.concatenate([lg_ref[...], lg_ref[...]], axis=1)
    gamma8 = jnp.exp(lg8)[0][:, None][None, :, :]
    distp = jnp.stack(
        [jnp.concatenate(
            [jnp.broadcast_to(dist_m[2 * q:2 * q + 1], (4, n)),
             jnp.broadcast_to(dist_m[2 * q + 1:2 * q + 2], (4, n))], axis=0)
         for q in range(hp)], axis=0)
    e_log = -distp * gamma8
    s_log = s_log - _INF * eyep

    e_exp = jnp.exp(e_log)
    ze = jnp.sum(e_exp, axis=-1, keepdims=True)
    ms = jnp.max(s_log, axis=-1, keepdims=True)
    s_exp = jnp.exp(s_log - ms)
    zs = jnp.sum(s_exp, axis=-1, keepdims=True)
    u = e_exp * s_exp
    comb = u / (jnp.sum(u, axis=-1, keepdims=True) + _EPS * ze * zs)

    w = 1.0 / (dist + _EPS)
    wdx0 = w * dx0
    wdx1 = w * dx1
    wdx2 = w * dx2
    qs_rows = []
    for b in range(blk):
        g_all = jax.lax.dot_general(
            wg_ref[...], h_e[b], (((1,), (0,)), ((), ())),
            preferred_element_type=jnp.float32)
        r0 = 4 * (b % 2)
        acc = comb[b // 2, r0:r0 + 1, :] * g_all[0:32, :]
        for m in range(1, 4):
            acc = acc + (comb[b // 2, r0 + m:r0 + m + 1, :]
                         * g_all[32 * m:32 * (m + 1), :])
        qs_rows.append(acc)
    n1_agg = jnp.sum(jnp.stack(qs_rows, axis=0), axis=-1)
    csn0 = jnp.sum(h_e * wdx0[:, None, :], axis=-1)
    csn1 = jnp.sum(h_e * wdx1[:, None, :], axis=-1)
    csn2 = jnp.sum(h_e * wdx2[:, None, :], axis=-1)

    inv_n = 1.0 / n
    cnorm = (csn0 * csn0 + csn1 * csn1 + csn2 * csn2) * (inv_n * inv_n)
    hc1 = _silu(jnp.dot(cnorm, pnw1_ref[...],
                        preferred_element_type=jnp.float32) + pnb1_ref[...])
    h_comb = _silu(jnp.dot(hc1, pnw2_ref[...],
                           preferred_element_type=jnp.float32) + pnb2_ref[...])

    vm = vmix_ref[...]
    dv0 = jnp.dot(csn0, vm, preferred_element_type=jnp.float32) * inv_n
    dv1 = jnp.dot(csn1, vm, preferred_element_type=jnp.float32) * inv_n
    dv2 = jnp.dot(csn2, vm, preferred_element_type=jnp.float32) * inv_n
    xout_ref[...] = x_blk + jnp.concatenate([dv0, dv1, dv2], axis=1)

    hvh = nw1_ref.shape[0]
    n1 = (jnp.dot(h_blk, nw1_ref[0:d_in, :],
                  preferred_element_type=jnp.float32)
          + jnp.dot(h_comb, nw1_ref[hvh - 32:hvh, :],
                    preferred_element_type=jnp.float32)
          + n1_agg + nb1_ref[...])
    n1 = _silu(n1)
    hout_ref[...] = h_blk + _silu(
        jnp.dot(n1, nw2_ref[...], preferred_element_type=jnp.float32)
        + nb2_ref[...])


def kernel(h, x, edge_w1, edge_b1, edge_w2, edge_b2, sem_w, sem_b, pn_w1,
           pn_b1, pn_w2, pn_b2, node_w1, node_b1, node_w2, node_b2, vmix_w,
           log_gamma):
    n, d_in = h.shape
    hidden = edge_w2.shape[0]
    heads = sem_w.shape[1]
    d_out = node_w2.shape[1]
    blk = _BLK

    b1 = edge_b1.reshape(1, hidden)
    b2c = edge_b2.reshape(hidden, 1)
    sembc = sem_b.reshape(heads, 1)
    pnb1 = pn_b1.reshape(1, hidden)
    pnb2 = pn_b2.reshape(1, hidden)
    wg = (node_w1[d_in:d_in + heads * hidden]
          .reshape(hidden, heads, hidden).transpose(1, 2, 0)
          .reshape(heads * hidden, hidden))
    nb1 = node_b1.reshape(1, hidden)
    nb2 = node_b2.reshape(1, d_out)
    lg = log_gamma.reshape(1, heads)

    full = lambda shape: pl.BlockSpec(shape, lambda i: (0,) * len(shape))
    out = pl.pallas_call(
        _sake_body,
        grid=(n // blk,),
        compiler_params=pltpu.CompilerParams(
            dimension_semantics=("parallel",)),
        in_specs=[
            pl.BlockSpec((blk, d_in), lambda i: (i, 0)),
            pl.BlockSpec((blk, 3), lambda i: (i, 0)),
            full((n, d_in)),
            full((n, 3)),
            full((2 * d_in + 1, hidden)),
            full((1, hidden)),
            full((hidden, hidden)),
            full((hidden, 1)),
            full((hidden, heads)),
            full((heads, 1)),
            full((hidden, hidden)),
            full((1, hidden)),
            full((hidden, hidden)),
            full((1, hidden)),
            full((heads * hidden + hidden + d_in, hidden)),
            full((heads * hidden, hidden)),
            full((1, hidden)),
            full((hidden, d_out)),
            full((1, d_out)),
            full((hidden, 1)),
            full((1, heads)),
        ],
        out_specs=[
            pl.BlockSpec((blk, d_out), lambda i: (i, 0)),
            pl.BlockSpec((blk, 3), lambda i: (i, 0)),
        ],
        out_shape=[
            jax.ShapeDtypeStruct((n, d_out), jnp.float32),
            jax.ShapeDtypeStruct((n, 3), jnp.float32),
        ],
        scratch_shapes=[pltpu.VMEM((hidden, n), jnp.float32),
                        pltpu.VMEM((3, n), jnp.float32)],
    )(h, x, h, x, edge_w1, b1, edge_w2, b2c, sem_w, sembc,
      pn_w1, pnb1, pn_w2, pnb2, node_w1, wg, nb1, node_w2, nb2,
      vmix_w, lg)
    return out[0], out[1]
